# baseline (speedup 1.0000x reference)
import functools

import jax
import jax.numpy as jnp
from jax import lax
from jax.experimental import pallas as pl
from jax.experimental.pallas import tpu as pltpu
from jax.experimental.pallas import tpu_sc as plsc

N_NODES = 10000
N_EDGES = 160000
DIM = 256
HALF = DIM // 2
NUM_HOPS = 3
MIN_NORM = 1e-15
EPS = 1e-7

NC = 2
NS = 16
LANES = 16

CHUNK = 128
NB = 2
NE = 4
NCH = 80
EPT = NCH * CHUNK
E_PAD = EPT * NS
N_PAD = 10240
RPT = N_PAD // NS
NZ = RPT // CHUNK


def _pre_body(w_ref, o_ref):
    w = w_ref[...]
    d = w[:, 1:]
    y2 = jnp.sum(d * d, axis=1, keepdims=True)
    x0 = jnp.sqrt(jnp.clip(1.0 + y2, EPS, None))
    y_norm = jnp.clip(jnp.sqrt(y2), MIN_NORM, None)
    theta = jnp.clip(x0, 1.0 + EPS, None)
    r = jnp.log(theta + jnp.sqrt(theta * theta - 1.0))
    res = (r / y_norm) * d
    xt = jnp.concatenate([jnp.zeros_like(w[:, :1]), res], axis=1)
    o_ref[0] = xt[:, :HALF]
    o_ref[1] = xt[:, HALF:]


def _pre(weight):
    rows = 1000
    return pl.pallas_call(
        _pre_body,
        grid=(N_NODES // rows,),
        in_specs=[pl.BlockSpec((rows, DIM), lambda i: (i, 0))],
        out_specs=pl.BlockSpec((2, rows, HALF), lambda i: (0, i, 0)),
        out_shape=jax.ShapeDtypeStruct((2, N_NODES, HALF), jnp.float32),
    )(weight)


def _post_body(h_ref, o_ref):
    h = h_ref[...]
    acc = h[1] + h[2] + h[3]
    u = jnp.concatenate([acc[0], acc[1]], axis=1)
    d = u[:, 1:]
    x_norm = jnp.clip(jnp.sqrt(jnp.sum(d * d, axis=1, keepdims=True)),
                      MIN_NORM, None)
    sinh = 0.5 * (jnp.exp(x_norm) - jnp.exp(-x_norm))
    rest = sinh * d / x_norm
    y2 = jnp.sum(rest * rest, axis=1, keepdims=True)
    x0 = jnp.sqrt(jnp.clip(1.0 + y2, EPS, None))
    o_ref[...] = jnp.concatenate([x0, rest], axis=1)


def _post(hs):
    rows = 1000
    return pl.pallas_call(
        _post_body,
        grid=(N_NODES // rows,),
        in_specs=[pl.BlockSpec((NUM_HOPS + 1, 2, rows, HALF),
                               lambda i: (0, 0, i, 0))],
        out_specs=pl.BlockSpec((rows, DIM), lambda i: (i, 0)),
        out_shape=jax.ShapeDtypeStruct((N_NODES, DIM), jnp.float32),
    )(hs)


def _sc_body(h0, edata, out, ering, bufs, esems, gsems, ssems, zsem, acc_sh):
    c = lax.axis_index("c")
    s = lax.axis_index("s")
    zeros16 = jnp.zeros((LANES,), jnp.float32)
    rbase = s * NCH

    def istage(sl, ch):
        pltpu.async_copy(edata.at[rbase + ch], ering.at[sl], esems.at[sl])

    def iwait(sl, ch):
        pltpu.make_async_copy(edata.at[rbase + ch], ering.at[sl],
                              esems.at[sl]).wait()

    def scale_chunk(db, sl):
        buf = bufs.at[db]
        wrow = ering.at[sl, 2]

        def body16(e16, _):
            wv = lax.bitcast_convert_type(wrow[pl.ds(e16 * LANES, LANES)],
                                          jnp.float32)
            for k in range(LANES):
                w = wv[k]
                e = e16 * LANES + k
                for j in range(HALF // LANES):
                    v = buf[e, pl.ds(j * LANES, LANES)]
                    buf[e, pl.ds(j * LANES, LANES)] = v * w
            return 0
        lax.fori_loop(0, CHUNK // LANES, body16, 0)

    @pl.when(s < NS - 1)
    def _():
        pltpu.sync_copy(h0.at[c, pl.ds(s * RPT, RPT)],
                        out.at[0, c, pl.ds(s * RPT, RPT)])

    @pl.when(s == NS - 1)
    def _():
        last = N_NODES - (NS - 1) * RPT
        pltpu.sync_copy(h0.at[c, pl.ds((NS - 1) * RPT, last)],
                        out.at[0, c, pl.ds((NS - 1) * RPT, last)])

    def hop_body(hop, _):
        hsrc = out.at[hop, c]

        def gstart(db, sl):
            pltpu.async_copy(hsrc.at[ering.at[sl, 0]], bufs.at[db],
                             gsems.at[db])

        def gwait(db, sl):
            pltpu.make_async_copy(hsrc.at[ering.at[sl, 0]], bufs.at[db],
                                  gsems.at[db]).wait()

        def sstart(db, sl):
            pltpu.async_copy(bufs.at[db], acc_sh.at[ering.at[sl, 1]],
                             ssems.at[db], add=True)

        def swait(db, sl):
            pltpu.make_async_copy(bufs.at[db], acc_sh.at[ering.at[sl, 1]],
                                  ssems.at[db]).wait()

        def zbody(e, _):
            for j in range(HALF // LANES):
                bufs[0, e, pl.ds(j * LANES, LANES)] = zeros16
            return 0
        lax.fori_loop(0, CHUNK, zbody, 0)
        for z in range(NZ):
            pltpu.async_copy(bufs.at[0],
                             acc_sh.at[pl.ds(s * RPT + z * CHUNK, CHUNK)],
                             zsem)
        for b in range(NE):
            istage(b, b)
        for z in range(NZ):
            pltpu.make_async_copy(
                bufs.at[0], acc_sh.at[pl.ds(s * RPT + z * CHUNK, CHUNK)],
                zsem).wait()
        plsc.subcore_barrier()

        def step(i, _):
            b2 = lax.rem(i, NB)
            e4 = lax.rem(i, NE)
            pe4 = lax.rem(i + NB, NE)

            @pl.when(i >= NB)
            def _():
                swait(b2, pe4)

                @pl.when(i + NB < NCH)
                def _():
                    istage(pe4, i + NB)

            @pl.when(i < NCH)
            def _():
                iwait(e4, i)
                gstart(b2, e4)

            j = i - 1
            jb2 = lax.rem(j + NB, NB)
            je4 = lax.rem(j + NE, NE)

            @pl.when(jnp.logical_and(j >= 0, j < NCH))
            def _():
                gwait(jb2, je4)
                scale_chunk(jb2, je4)
                sstart(jb2, je4)
            return 0

        lax.fori_loop(0, NCH + NB, step, 0)
        plsc.subcore_barrier()

        pltpu.sync_copy(acc_sh.at[pl.ds(s * RPT, RPT)],
                        out.at[hop + 1, c, pl.ds(s * RPT, RPT)])
        plsc.subcore_barrier()
        return 0

    lax.fori_loop(0, NUM_HOPS, hop_body, 0)


def _spmm(xt2, edata):
    mesh = plsc.VectorSubcoreMesh(core_axis_name="c", subcore_axis_name="s")
    f = functools.partial(
        pl.kernel,
        mesh=mesh,
        out_type=jax.ShapeDtypeStruct((NUM_HOPS + 1, 2, N_PAD, HALF),
                                      jnp.float32),
        scratch_types=[
            pltpu.VMEM((NE, 3, CHUNK), jnp.int32),
            pltpu.VMEM((NB, CHUNK, HALF), jnp.float32),
            pltpu.SemaphoreType.DMA((NE,)),
            pltpu.SemaphoreType.DMA((NB,)),
            pltpu.SemaphoreType.DMA((NB,)),
            pltpu.SemaphoreType.DMA,
            pltpu.VMEM_SHARED((N_PAD, HALF), jnp.float32),
        ],
    )(_sc_body)
    return f(xt2, edata)


def kernel(weight, edge_index, edge_weight):
    xt2 = _pre(weight)
    pad = E_PAD - N_EDGES
    srcs = jnp.concatenate([edge_index[0], jnp.zeros((pad,), jnp.int32)])
    dsts = jnp.concatenate([edge_index[1], jnp.zeros((pad,), jnp.int32)])
    ws = jnp.concatenate([edge_weight, jnp.zeros((pad,), jnp.float32)])
    edata = jnp.stack(
        [srcs.reshape(NS * NCH, CHUNK), dsts.reshape(NS * NCH, CHUNK),
         lax.bitcast_convert_type(ws, jnp.int32).reshape(NS * NCH, CHUNK)],
        axis=1)
    hs = _spmm(xt2, edata)
    return _post(hs)


# speedup vs baseline: 1.4192x; 1.4192x over previous
import functools

import jax
import jax.numpy as jnp
from jax import lax
from jax.experimental import pallas as pl
from jax.experimental.pallas import tpu as pltpu
from jax.experimental.pallas import tpu_sc as plsc

N_NODES = 10000
N_EDGES = 160000
DIM = 256
HALF = DIM // 2
NUM_HOPS = 3
MIN_NORM = 1e-15
EPS = 1e-7

NC = 2
NS = 16
LANES = 16

CHUNK = 128
NB = 2
NE = 4
NCH = 80
NG = NCH // NE
EPT = NCH * CHUNK
E_PAD = EPT * NS
N_PAD = 10240
RPT = N_PAD // NS
NZ = RPT // CHUNK


def _pre_body(w_ref, o_ref):
    w = w_ref[...]
    d = w[:, 1:]
    y2 = jnp.sum(d * d, axis=1, keepdims=True)
    x0 = jnp.sqrt(jnp.clip(1.0 + y2, EPS, None))
    y_norm = jnp.clip(jnp.sqrt(y2), MIN_NORM, None)
    theta = jnp.clip(x0, 1.0 + EPS, None)
    r = jnp.log(theta + jnp.sqrt(theta * theta - 1.0))
    res = (r / y_norm) * d
    xt = jnp.concatenate([jnp.zeros_like(w[:, :1]), res], axis=1)
    o_ref[0] = xt[:, :HALF]
    o_ref[1] = xt[:, HALF:]


def _pre(weight):
    rows = 1000
    return pl.pallas_call(
        _pre_body,
        grid=(N_NODES // rows,),
        in_specs=[pl.BlockSpec((rows, DIM), lambda i: (i, 0))],
        out_specs=pl.BlockSpec((2, rows, HALF), lambda i: (0, i, 0)),
        out_shape=jax.ShapeDtypeStruct((2, N_NODES, HALF), jnp.float32),
    )(weight)


def _post_body(h_ref, o_ref):
    h = h_ref[...]
    acc = h[1] + h[2] + h[3]
    u = jnp.concatenate([acc[0], acc[1]], axis=1)
    d = u[:, 1:]
    x_norm = jnp.clip(jnp.sqrt(jnp.sum(d * d, axis=1, keepdims=True)),
                      MIN_NORM, None)
    sinh = 0.5 * (jnp.exp(x_norm) - jnp.exp(-x_norm))
    rest = sinh * d / x_norm
    y2 = jnp.sum(rest * rest, axis=1, keepdims=True)
    x0 = jnp.sqrt(jnp.clip(1.0 + y2, EPS, None))
    o_ref[...] = jnp.concatenate([x0, rest], axis=1)


def _post(hs):
    rows = 1000
    return pl.pallas_call(
        _post_body,
        grid=(N_NODES // rows,),
        in_specs=[pl.BlockSpec((NUM_HOPS + 1, 2, rows, HALF),
                               lambda i: (0, 0, i, 0))],
        out_specs=pl.BlockSpec((rows, DIM), lambda i: (i, 0)),
        out_shape=jax.ShapeDtypeStruct((N_NODES, DIM), jnp.float32),
    )(hs)


def _sc_body(h0, edata, out, ering, bufs, esems, gsems, ssems, zsem, acc_sh):
    c = lax.axis_index("c")
    s = lax.axis_index("s")
    zeros16 = jnp.zeros((LANES,), jnp.float32)
    rbase = s * NCH

    def istage(sl, ch):
        pltpu.async_copy(edata.at[rbase + ch], ering.at[sl], esems.at[sl])

    def iwait(sl, ch):
        pltpu.make_async_copy(edata.at[rbase + ch], ering.at[sl],
                              esems.at[sl]).wait()

    def scale_chunk(db, sl):
        buf = bufs.at[db]
        wrow = ering.at[sl, 2]

        def body16(e16, _):
            wv = lax.bitcast_convert_type(wrow[pl.ds(e16 * LANES, LANES)],
                                          jnp.float32)
            for k in range(LANES):
                w = wv[k]
                e = e16 * LANES + k
                for j in range(HALF // LANES):
                    v = buf[e, pl.ds(j * LANES, LANES)]
                    buf[e, pl.ds(j * LANES, LANES)] = v * w
            return 0
        lax.fori_loop(0, CHUNK // LANES, body16, 0)

    @pl.when(s < NS - 1)
    def _():
        pltpu.sync_copy(h0.at[c, pl.ds(s * RPT, RPT)],
                        out.at[0, c, pl.ds(s * RPT, RPT)])

    @pl.when(s == NS - 1)
    def _():
        last = N_NODES - (NS - 1) * RPT
        pltpu.sync_copy(h0.at[c, pl.ds((NS - 1) * RPT, last)],
                        out.at[0, c, pl.ds((NS - 1) * RPT, last)])

    def hop_body(hop, _):
        hsrc = out.at[hop, c]

        def gstart(db, sl):
            pltpu.async_copy(hsrc.at[ering.at[sl, 0]], bufs.at[db],
                             gsems.at[db])

        def gwait(db, sl):
            pltpu.make_async_copy(hsrc.at[ering.at[sl, 0]], bufs.at[db],
                                  gsems.at[db]).wait()

        def sstart(db, sl):
            pltpu.async_copy(bufs.at[db], acc_sh.at[ering.at[sl, 1]],
                             ssems.at[db], add=True)

        def swait(db, sl):
            pltpu.make_async_copy(bufs.at[db], acc_sh.at[ering.at[sl, 1]],
                                  ssems.at[db]).wait()

        def step(b, i, warm, stage):
            if not warm:
                swait(b % NB, (b - 2) % NE)
                if stage:
                    istage((b - 2) % NE, i + NB)
            iwait(b, i)
            gstart(b % NB, b)
            sj, dj = (b - 1) % NE, (b - 1) % NB
            if not (warm and b == 0):
                gwait(dj, sj)
                scale_chunk(dj, sj)
                sstart(dj, sj)

        def zbody(e, _):
            for j in range(HALF // LANES):
                bufs[0, e, pl.ds(j * LANES, LANES)] = zeros16
            return 0
        lax.fori_loop(0, CHUNK, zbody, 0)
        for z in range(NZ):
            pltpu.async_copy(bufs.at[0],
                             acc_sh.at[pl.ds(s * RPT + z * CHUNK, CHUNK)],
                             zsem)
        for b in range(NE):
            istage(b, b)
        for z in range(NZ):
            pltpu.make_async_copy(
                bufs.at[0], acc_sh.at[pl.ds(s * RPT + z * CHUNK, CHUNK)],
                zsem).wait()
        plsc.subcore_barrier()

        step(0, 0, True, False)
        step(1, 1, True, False)
        step(2, 2, False, True)
        step(3, 3, False, True)

        def group(g, _):
            for b in range(NE):
                step(b, g * NE + b, False, True)
            return 0
        lax.fori_loop(1, NG - 1, group, 0)
        for b in range(NE):
            i = (NG - 1) * NE + b
            step(b, i, False, i + NB < NCH)
        j = NCH - 1
        gwait(j % NB, j % NE)
        scale_chunk(j % NB, j % NE)
        sstart(j % NB, j % NE)
        swait((NCH - 2) % NB, (NCH - 2) % NE)
        swait((NCH - 1) % NB, (NCH - 1) % NE)
        plsc.subcore_barrier()

        pltpu.sync_copy(acc_sh.at[pl.ds(s * RPT, RPT)],
                        out.at[hop + 1, c, pl.ds(s * RPT, RPT)])
        plsc.subcore_barrier()
        return 0

    lax.fori_loop(0, NUM_HOPS, hop_body, 0)


def _spmm(xt2, edata):
    mesh = plsc.VectorSubcoreMesh(core_axis_name="c", subcore_axis_name="s")
    f = functools.partial(
        pl.kernel,
        mesh=mesh,
        out_type=jax.ShapeDtypeStruct((NUM_HOPS + 1, 2, N_PAD, HALF),
                                      jnp.float32),
        scratch_types=[
            pltpu.VMEM((NE, 3, CHUNK), jnp.int32),
            pltpu.VMEM((NB, CHUNK, HALF), jnp.float32),
            pltpu.SemaphoreType.DMA((NE,)),
            pltpu.SemaphoreType.DMA((NB,)),
            pltpu.SemaphoreType.DMA((NB,)),
            pltpu.SemaphoreType.DMA,
            pltpu.VMEM_SHARED((N_PAD, HALF), jnp.float32),
        ],
    )(_sc_body)
    return f(xt2, edata)


def kernel(weight, edge_index, edge_weight):
    xt2 = _pre(weight)
    pad = E_PAD - N_EDGES
    srcs = jnp.concatenate([edge_index[0], jnp.zeros((pad,), jnp.int32)])
    dsts = jnp.concatenate([edge_index[1], jnp.zeros((pad,), jnp.int32)])
    ws = jnp.concatenate([edge_weight, jnp.zeros((pad,), jnp.float32)])
    edata = jnp.stack(
        [srcs.reshape(NS * NCH, CHUNK), dsts.reshape(NS * NCH, CHUNK),
         lax.bitcast_convert_type(ws, jnp.int32).reshape(NS * NCH, CHUNK)],
        axis=1)
    hs = _spmm(xt2, edata)
    return _post(hs)
